# baseline (speedup 1.0000x reference)
import math

import jax
import jax.numpy as jnp
from jax import lax
from jax.experimental import pallas as pl
from jax.experimental.pallas import tpu as pltpu

_B, _LX, _LQ = 4, 256, 64
_D, _LAT, _NL, _NH, _NG = 41, 128, 2, 4, 2
_L = _LX + _LQ
_DP = 48
_DH = _LAT // _NH
_HALF = _LAT // 2
_SC = math.sqrt(_DH)
_NEG = -1e9


def _fwd_kernel(tx_ref, cx_ref, mx_ref, x_ref, tq_ref, cq_ref, mq_ref,
                W_time_ref, C_emb_ref, W_e_ref,
                Wq_c_ref, Wk_c_ref, Wv_c_ref, Wo_c_ref,
                Wq_t_ref, Wk_t_ref, Wv_t_ref, Wo_t_ref,
                W_edge_ref, W_out_ref,
                h_obs_ref, h_qry_ref):
    f32 = jnp.float32
    tx = tx_ref[0]
    tq = tq_ref[0]
    x = x_ref[0]
    mx = mx_ref[0]
    mq = mq_ref[0]
    cx = cx_ref[0]
    cq = cq_ref[0]

    dio_x = lax.broadcasted_iota(jnp.int32, (_DP, _LX), 0)
    ohxT = (dio_x == cx).astype(f32)
    dio_q = lax.broadcasted_iota(jnp.int32, (_DP, _LQ), 0)
    ohqT = (dio_q == cq).astype(f32)

    xv_T = ohxT * (x * mx)
    xm_T = ohxT * mx
    ym_T = ohqT * mq

    bigT = jnp.transpose(jnp.concatenate([xv_T, xm_T, ym_T], axis=1))
    x_vals_o = bigT[:_LX]
    x_mask_o = bigT[_LX:2 * _LX]
    y_mask_q = bigT[2 * _LX:]

    zeros_q = jnp.zeros((_LQ, _DP), f32)
    zeros_o = jnp.zeros((_LX, _DP), f32)
    x_vals = jnp.concatenate([x_vals_o, zeros_q], axis=0)
    x_mask = jnp.concatenate([x_mask_o, zeros_q], axis=0)
    y_mask = jnp.concatenate([zeros_o, y_mask_q], axis=0)
    maskf = x_mask + y_mask
    biasf = jnp.where(maskf > 0.0, 0.0, _NEG)
    bias3 = jnp.broadcast_to(biasf[:, :, None], (_L, _DP, _LAT))

    t_all = jnp.concatenate([tx, tq], axis=1)
    fio = lax.broadcasted_iota(jnp.int32, (_HALF, _L), 0).astype(f32)
    freqs = jnp.exp(fio * (8.0 / (_HALF - 1)))
    tf = freqs * t_all
    featsT = jnp.concatenate([jnp.sin(tf), jnp.cos(tf)], axis=0)
    tm = lax.dot_general(featsT, W_time_ref[...],
                         (((0,), (0,)), ((), ())),
                         preferred_element_type=f32)

    hio_r = lax.shift_right_logical(
        lax.broadcasted_iota(jnp.int32, (_LAT, _LAT), 0), 5)
    hio_c = lax.shift_right_logical(
        lax.broadcasted_iota(jnp.int32, (_LAT, _LAT), 1), 5)
    BD = (hio_r == hio_c).astype(f32)

    SEL2 = (lax.broadcasted_iota(jnp.int32, (_L, _DP, _DP), 1)
            == lax.broadcasted_iota(jnp.int32, (_L, _DP, _DP), 2)
            ).astype(f32).reshape(_L * _DP, _DP)
    bias2 = bias3.reshape(_L * _DP, _LAT)

    We = W_e_ref[...]
    We0 = We[0:1, :].reshape(1, 1, _LAT)
    We1 = We[1:2, :].reshape(1, 1, _LAT)
    We2 = We[2:3, :].reshape(1, 1, _LAT)
    edge = (x_vals[:, :, None] * We0
            + x_mask[:, :, None] * We1
            + y_mask[:, :, None] * We2)

    ch = C_emb_ref[...]

    for l in range(_NL):
        Wq = Wq_c_ref[l]
        Wk = Wk_c_ref[l]
        Wv = Wv_c_ref[l]
        Wo = Wo_c_ref[l]
        Wqt = Wq_t_ref[l]
        Wkt = Wk_t_ref[l]
        Wvt = Wv_t_ref[l]
        Wed = W_edge_ref[l]
        e2 = edge.reshape(_L * _DP, _LAT)
        e_kv = jnp.dot(e2, jnp.concatenate([Wk[:_LAT], Wv[:_LAT]], axis=1),
                       preferred_element_type=f32)
        tm3 = jnp.dot(
            tm,
            jnp.concatenate([Wk[_LAT:], Wv[_LAT:], Wqt], axis=1),
            preferred_element_type=f32)
        q = jnp.dot(ch, Wq, preferred_element_type=f32) * (1.0 / _SC)
        p = ((e_kv[:, :_LAT].reshape(_L, _DP, _LAT) + tm3[:, None, :_LAT])
             * q[None, :, :]).reshape(_L * _DP, _LAT)
        e_s = jnp.exp(jnp.dot(p, BD, preferred_element_type=f32) + bias2)
        ew = (e_s.reshape(_L, _DP, _LAT)
              * (e_kv[:, _LAT:].reshape(_L, _DP, _LAT)
                 + tm3[:, None, _LAT:2 * _LAT])).reshape(_L * _DP, _LAT)
        sum_s = lax.dot_general(SEL2, e_s, (((0,), (0,)), ((), ())),
                                preferred_element_type=f32)
        o_raw = lax.dot_general(SEL2, ew, (((0,), (0,)), ((), ())),
                                preferred_element_type=f32)
        o = o_raw * (1.0 / (sum_s + 1e-30))
        ch = ch + jax.nn.relu(jnp.dot(o, Wo, preferred_element_type=f32))

        Wot = Wo_t_ref[l]
        qt = tm3[:, 2 * _LAT:] * (1.0 / _SC)
        ch3 = jnp.dot(
            ch,
            jnp.concatenate([Wkt[_LAT:], Wvt[_LAT:], Wed[2 * _LAT:]], axis=1),
            preferred_element_type=f32)
        e_tt = jnp.dot(e2, jnp.concatenate([Wkt[:_LAT], Wvt[:_LAT]], axis=1),
                       preferred_element_type=f32)
        pt = ((e_tt[:, :_LAT].reshape(_L, _DP, _LAT) + ch3[None, :, :_LAT])
              * qt[:, None, :]).reshape(_L * _DP, _LAT)
        st_rep = (jnp.dot(pt, BD, preferred_element_type=f32)
                  .reshape(_L, _DP, _LAT) + bias3)
        e_t = jnp.exp(st_rep)
        r_t = 1.0 / (jnp.sum(e_t, axis=1) + 1e-30)
        ot = jnp.sum(e_t * (e_tt[:, _LAT:].reshape(_L, _DP, _LAT)
                            + ch3[None, :, _LAT:2 * _LAT]),
                     axis=1) * r_t
        tm = tm + jax.nn.relu(jnp.dot(ot, Wot, preferred_element_type=f32))

        pre = (jnp.dot(e2, Wed[:_LAT], preferred_element_type=f32)
               .reshape(_L, _DP, _LAT)
               + jnp.dot(tm, Wed[_LAT:2 * _LAT],
                         preferred_element_type=f32)[:, None, :]
               + ch3[None, :, 2 * _LAT:])
        edge = edge + jax.nn.relu(pre)

    ohq = jnp.transpose(ohqT)
    edge_q = jnp.sum(edge[_LX:, :, :] * ohq[:, :, None], axis=1)
    tm_q = tm[_LX:, :]
    ch_q = jnp.dot(ohq, ch, preferred_element_type=f32)
    Wout = W_out_ref[...]
    hq = (jnp.dot(edge_q, Wout[:_LAT], preferred_element_type=f32)
          + jnp.dot(tm_q, Wout[_LAT:2 * _LAT], preferred_element_type=f32)
          + jnp.dot(ch_q, Wout[2 * _LAT:], preferred_element_type=f32))
    h_qry_ref[0] = hq

    ohx_m = jnp.transpose(ohxT * mx)
    h_obs = jnp.sum(edge[:_LX, :, :] * ohx_m[:, :, None], axis=1)
    h_obs_ref[0] = h_obs


def kernel(tx, cx, mx, x, tq, cq, mq, W_time, C_emb, W_e, Wq_c, Wk_c, Wv_c,
           Wo_c, Wq_t, Wk_t, Wv_t, Wo_t, W_edge, W_out):
    f32 = jnp.float32
    txr = tx.reshape(_B, 1, _LX).astype(f32)
    cxr = cx.reshape(_B, 1, _LX).astype(jnp.int32)
    mxr = mx.reshape(_B, 1, _LX).astype(f32)
    xr = x.reshape(_B, 1, _LX).astype(f32)
    tqr = tq.reshape(_B, 1, _LQ).astype(f32)
    cqr = cq.reshape(_B, 1, _LQ).astype(jnp.int32)
    mqr = mq.reshape(_B, 1, _LQ).astype(f32)
    C_emb_p = jnp.zeros((_DP, _LAT), f32).at[:_D].set(C_emb.astype(f32))

    def row_spec(n):
        return pl.BlockSpec((1, 1, n), lambda b: (b, 0, 0))

    def full_spec(arr):
        nd = arr.ndim
        return pl.BlockSpec(arr.shape, lambda b: (0,) * nd)

    weights = [W_time, C_emb_p, W_e, Wq_c, Wk_c, Wv_c, Wo_c,
               Wq_t, Wk_t, Wv_t, Wo_t, W_edge, W_out]
    in_specs = [row_spec(_LX), row_spec(_LX), row_spec(_LX), row_spec(_LX),
                row_spec(_LQ), row_spec(_LQ), row_spec(_LQ)]
    in_specs += [full_spec(w) for w in weights]

    out_shape = [jax.ShapeDtypeStruct((_B, _LX, _LAT), f32),
                 jax.ShapeDtypeStruct((_B, _LQ, _NG * _LAT), f32)]
    out_specs = [pl.BlockSpec((1, _LX, _LAT), lambda b: (b, 0, 0)),
                 pl.BlockSpec((1, _LQ, _NG * _LAT), lambda b: (b, 0, 0))]

    h_obs, hq = pl.pallas_call(
        _fwd_kernel,
        grid=(_B,),
        in_specs=in_specs,
        out_specs=out_specs,
        out_shape=out_shape,
        compiler_params=pltpu.CompilerParams(
            dimension_semantics=("parallel",),
            vmem_limit_bytes=128 * 1024 * 1024,
        ),
    )(txr, cxr, mxr, xr, tqr, cqr, mqr, *weights)

    h_qry = hq.reshape(_B, _LQ, _NG, _LAT).transpose(0, 2, 1, 3)
    return h_obs, h_qry


# speedup vs baseline: 1.1226x; 1.1226x over previous
import math

import jax
import jax.numpy as jnp
from jax import lax
from jax.experimental import pallas as pl
from jax.experimental.pallas import tpu as pltpu

_B, _LX, _LQ = 4, 256, 64
_D, _LAT, _NL, _NH, _NG = 41, 128, 2, 4, 2
_L = _LX + _LQ
_DP = 48
_DH = _LAT // _NH
_HALF = _LAT // 2
_SC = math.sqrt(_DH)
_NEG = -1e9


def _fwd_kernel(tx_ref, cx_ref, mx_ref, x_ref, tq_ref, cq_ref, mq_ref,
                W_time_ref, C_emb_ref, W_e_ref,
                Wq_c_ref, Wk_c_ref, Wv_c_ref, Wo_c_ref,
                Wq_t_ref, Wk_t_ref, Wv_t_ref, Wo_t_ref,
                W_edge_ref, W_out_ref,
                h_obs_ref, h_qry_ref):
    f32 = jnp.float32
    tx = tx_ref[0]
    tq = tq_ref[0]
    x = x_ref[0]
    mx = mx_ref[0]
    mq = mq_ref[0]
    cx = cx_ref[0]
    cq = cq_ref[0]

    dio_x = lax.broadcasted_iota(jnp.int32, (_DP, _LX), 0)
    ohxT = (dio_x == cx).astype(f32)
    dio_q = lax.broadcasted_iota(jnp.int32, (_DP, _LQ), 0)
    ohqT = (dio_q == cq).astype(f32)

    xv_T = ohxT * (x * mx)
    xm_T = ohxT * mx
    ym_T = ohqT * mq

    bigT = jnp.transpose(jnp.concatenate([xv_T, xm_T, ym_T], axis=1))
    x_vals_o = bigT[:_LX]
    x_mask_o = bigT[_LX:2 * _LX]
    y_mask_q = bigT[2 * _LX:]

    zeros_q = jnp.zeros((_LQ, _DP), f32)
    zeros_o = jnp.zeros((_LX, _DP), f32)
    x_vals = jnp.concatenate([x_vals_o, zeros_q], axis=0)
    x_mask = jnp.concatenate([x_mask_o, zeros_q], axis=0)
    y_mask = jnp.concatenate([zeros_o, y_mask_q], axis=0)
    maskf = x_mask + y_mask
    biasf = jnp.where(maskf > 0.0, 0.0, _NEG)
    bias3 = jnp.broadcast_to(biasf[:, :, None], (_L, _DP, _LAT))

    t_all = jnp.concatenate([tx, tq], axis=1)
    fio = lax.broadcasted_iota(jnp.int32, (_HALF, _L), 0).astype(f32)
    freqs = jnp.exp(fio * (8.0 / (_HALF - 1)))
    tf = freqs * t_all
    featsT = jnp.concatenate([jnp.sin(tf), jnp.cos(tf)], axis=0)
    tm = lax.dot_general(featsT, W_time_ref[...],
                         (((0,), (0,)), ((), ())),
                         preferred_element_type=f32)

    hio_r = lax.shift_right_logical(
        lax.broadcasted_iota(jnp.int32, (_LAT, _LAT), 0), 5)
    hio_c = lax.shift_right_logical(
        lax.broadcasted_iota(jnp.int32, (_LAT, _LAT), 1), 5)
    BD = (hio_r == hio_c).astype(f32)

    bias2 = bias3.reshape(_L * _DP, _LAT)

    We = W_e_ref[...]
    We0 = We[0:1, :].reshape(1, 1, _LAT)
    We1 = We[1:2, :].reshape(1, 1, _LAT)
    We2 = We[2:3, :].reshape(1, 1, _LAT)
    edge = (x_vals[:, :, None] * We0
            + x_mask[:, :, None] * We1
            + y_mask[:, :, None] * We2)

    ch = C_emb_ref[...]

    for l in range(_NL):
        Wq = Wq_c_ref[l]
        Wk = Wk_c_ref[l]
        Wv = Wv_c_ref[l]
        Wo = Wo_c_ref[l]
        Wqt = Wq_t_ref[l]
        Wkt = Wk_t_ref[l]
        Wvt = Wv_t_ref[l]
        Wed = W_edge_ref[l]
        e2 = edge.reshape(_L * _DP, _LAT)
        e_kv = jnp.dot(e2, jnp.concatenate([Wk[:_LAT], Wv[:_LAT]], axis=1),
                       preferred_element_type=f32)
        tm3 = jnp.dot(
            tm,
            jnp.concatenate([Wk[_LAT:], Wv[_LAT:], Wqt], axis=1),
            preferred_element_type=f32)
        q = jnp.dot(ch, Wq, preferred_element_type=f32) * (1.0 / _SC)
        p = ((e_kv[:, :_LAT].reshape(_L, _DP, _LAT) + tm3[:, None, :_LAT])
             * q[None, :, :]).reshape(_L * _DP, _LAT)
        e_s = (jnp.exp(jnp.dot(p, BD, preferred_element_type=f32) + bias2)
               .reshape(_L, _DP, _LAT))
        r_s = 1.0 / (jnp.sum(e_s, axis=0) + 1e-30)
        o = jnp.sum(e_s * (e_kv[:, _LAT:].reshape(_L, _DP, _LAT)
                           + tm3[:, None, _LAT:2 * _LAT]),
                    axis=0) * r_s
        ch = ch + jax.nn.relu(jnp.dot(o, Wo, preferred_element_type=f32))

        Wot = Wo_t_ref[l]
        qt = tm3[:, 2 * _LAT:] * (1.0 / _SC)
        ch3 = jnp.dot(
            ch,
            jnp.concatenate([Wkt[_LAT:], Wvt[_LAT:], Wed[2 * _LAT:]], axis=1),
            preferred_element_type=f32)
        e_tt = jnp.dot(e2, jnp.concatenate([Wkt[:_LAT], Wvt[:_LAT]], axis=1),
                       preferred_element_type=f32)
        pt = ((e_tt[:, :_LAT].reshape(_L, _DP, _LAT) + ch3[None, :, :_LAT])
              * qt[:, None, :]).reshape(_L * _DP, _LAT)
        e_t = (jnp.exp(jnp.dot(pt, BD, preferred_element_type=f32) + bias2)
               .reshape(_L, _DP, _LAT))
        r_t = 1.0 / (jnp.sum(e_t, axis=1) + 1e-30)
        ot = jnp.sum(e_t * (e_tt[:, _LAT:].reshape(_L, _DP, _LAT)
                            + ch3[None, :, _LAT:2 * _LAT]),
                     axis=1) * r_t
        tm = tm + jax.nn.relu(jnp.dot(ot, Wot, preferred_element_type=f32))

        pre = (jnp.dot(e2, Wed[:_LAT], preferred_element_type=f32)
               .reshape(_L, _DP, _LAT)
               + jnp.dot(tm, Wed[_LAT:2 * _LAT],
                         preferred_element_type=f32)[:, None, :]
               + ch3[None, :, 2 * _LAT:])
        edge = edge + jax.nn.relu(pre)

    ohq = jnp.transpose(ohqT)
    edge_q = jnp.sum(edge[_LX:, :, :] * ohq[:, :, None], axis=1)
    tm_q = tm[_LX:, :]
    ch_q = jnp.dot(ohq, ch, preferred_element_type=f32)
    Wout = W_out_ref[...]
    hq = (jnp.dot(edge_q, Wout[:_LAT], preferred_element_type=f32)
          + jnp.dot(tm_q, Wout[_LAT:2 * _LAT], preferred_element_type=f32)
          + jnp.dot(ch_q, Wout[2 * _LAT:], preferred_element_type=f32))
    h_qry_ref[0] = hq

    ohx_m = jnp.transpose(ohxT * mx)
    h_obs = jnp.sum(edge[:_LX, :, :] * ohx_m[:, :, None], axis=1)
    h_obs_ref[0] = h_obs


def kernel(tx, cx, mx, x, tq, cq, mq, W_time, C_emb, W_e, Wq_c, Wk_c, Wv_c,
           Wo_c, Wq_t, Wk_t, Wv_t, Wo_t, W_edge, W_out):
    f32 = jnp.float32
    txr = tx.reshape(_B, 1, _LX).astype(f32)
    cxr = cx.reshape(_B, 1, _LX).astype(jnp.int32)
    mxr = mx.reshape(_B, 1, _LX).astype(f32)
    xr = x.reshape(_B, 1, _LX).astype(f32)
    tqr = tq.reshape(_B, 1, _LQ).astype(f32)
    cqr = cq.reshape(_B, 1, _LQ).astype(jnp.int32)
    mqr = mq.reshape(_B, 1, _LQ).astype(f32)
    C_emb_p = jnp.zeros((_DP, _LAT), f32).at[:_D].set(C_emb.astype(f32))

    def row_spec(n):
        return pl.BlockSpec((1, 1, n), lambda b: (b, 0, 0))

    def full_spec(arr):
        nd = arr.ndim
        return pl.BlockSpec(arr.shape, lambda b: (0,) * nd)

    weights = [W_time, C_emb_p, W_e, Wq_c, Wk_c, Wv_c, Wo_c,
               Wq_t, Wk_t, Wv_t, Wo_t, W_edge, W_out]
    in_specs = [row_spec(_LX), row_spec(_LX), row_spec(_LX), row_spec(_LX),
                row_spec(_LQ), row_spec(_LQ), row_spec(_LQ)]
    in_specs += [full_spec(w) for w in weights]

    out_shape = [jax.ShapeDtypeStruct((_B, _LX, _LAT), f32),
                 jax.ShapeDtypeStruct((_B, _LQ, _NG * _LAT), f32)]
    out_specs = [pl.BlockSpec((1, _LX, _LAT), lambda b: (b, 0, 0)),
                 pl.BlockSpec((1, _LQ, _NG * _LAT), lambda b: (b, 0, 0))]

    h_obs, hq = pl.pallas_call(
        _fwd_kernel,
        grid=(_B,),
        in_specs=in_specs,
        out_specs=out_specs,
        out_shape=out_shape,
        compiler_params=pltpu.CompilerParams(
            dimension_semantics=("parallel",),
            vmem_limit_bytes=128 * 1024 * 1024,
        ),
    )(txr, cxr, mxr, xr, tqr, cqr, mqr, *weights)

    h_qry = hq.reshape(_B, _LQ, _NG, _LAT).transpose(0, 2, 1, 3)
    return h_obs, h_qry


# speedup vs baseline: 1.1290x; 1.0056x over previous
import math

import jax
import jax.numpy as jnp
from jax import lax
from jax.experimental import pallas as pl
from jax.experimental.pallas import tpu as pltpu

_B, _LX, _LQ = 4, 256, 64
_D, _LAT, _NL, _NH, _NG = 41, 128, 2, 4, 2
_L = _LX + _LQ
_DP = 48
_DH = _LAT // _NH
_HALF = _LAT // 2
_SC = math.sqrt(_DH)
_NEG = -1e9


def _fwd_kernel(tx_ref, cx_ref, mx_ref, x_ref, tq_ref, cq_ref, mq_ref,
                W_time_ref, C_emb_ref, W_e_ref,
                Wq_c_ref, Wk_c_ref, Wv_c_ref, Wo_c_ref,
                Wq_t_ref, Wk_t_ref, Wv_t_ref, Wo_t_ref,
                W_edge_ref, W_out_ref,
                h_obs_ref, h_qry_ref):
    f32 = jnp.float32
    tx = tx_ref[0]
    tq = tq_ref[0]
    x = x_ref[0]
    mx = mx_ref[0]
    mq = mq_ref[0]
    cx = cx_ref[0]
    cq = cq_ref[0]

    dio_x = lax.broadcasted_iota(jnp.int32, (_DP, _LX), 0)
    ohxT = (dio_x == cx).astype(f32)
    dio_q = lax.broadcasted_iota(jnp.int32, (_DP, _LQ), 0)
    ohqT = (dio_q == cq).astype(f32)

    xv_T = ohxT * (x * mx)
    xm_T = ohxT * mx
    ym_T = ohqT * mq

    bigT = jnp.transpose(jnp.concatenate([xv_T, xm_T, ym_T], axis=1))
    x_vals_o = bigT[:_LX]
    x_mask_o = bigT[_LX:2 * _LX]
    y_mask_q = bigT[2 * _LX:]

    zeros_q = jnp.zeros((_LQ, _DP), f32)
    zeros_o = jnp.zeros((_LX, _DP), f32)
    x_vals = jnp.concatenate([x_vals_o, zeros_q], axis=0)
    x_mask = jnp.concatenate([x_mask_o, zeros_q], axis=0)
    y_mask = jnp.concatenate([zeros_o, y_mask_q], axis=0)
    maskf = x_mask + y_mask
    biasf = jnp.where(maskf > 0.0, 0.0, _NEG)
    bias3 = jnp.broadcast_to(biasf[:, :, None], (_L, _DP, _LAT))

    t_all = jnp.concatenate([tx, tq], axis=1)
    fio = lax.broadcasted_iota(jnp.int32, (_HALF, _L), 0).astype(f32)
    freqs = jnp.exp(fio * (8.0 / (_HALF - 1)))
    tf = freqs * t_all
    featsT = jnp.concatenate([jnp.sin(tf), jnp.cos(tf)], axis=0)
    tm = lax.dot_general(featsT, W_time_ref[...],
                         (((0,), (0,)), ((), ())),
                         preferred_element_type=f32)

    hio_r = lax.shift_right_logical(
        lax.broadcasted_iota(jnp.int32, (_LAT, _LAT), 0), 5)
    hio_c = lax.shift_right_logical(
        lax.broadcasted_iota(jnp.int32, (_LAT, _LAT), 1), 5)
    BD = (hio_r == hio_c).astype(f32)

    bias2 = bias3.reshape(_L * _DP, _LAT)

    We = W_e_ref[...]
    We0 = We[0:1, :].reshape(1, 1, _LAT)
    We1 = We[1:2, :].reshape(1, 1, _LAT)
    We2 = We[2:3, :].reshape(1, 1, _LAT)
    edge_o = (x_vals_o[:, :, None] * We0
              + x_mask_o[:, :, None] * We1)
    edge_q0 = y_mask_q[:, :, None] * We2
    edge = jnp.concatenate([edge_o, edge_q0], axis=0)

    ch = C_emb_ref[...]

    for l in range(_NL):
        Wq = Wq_c_ref[l]
        Wk = Wk_c_ref[l]
        Wv = Wv_c_ref[l]
        Wo = Wo_c_ref[l]
        Wqt = Wq_t_ref[l]
        Wkt = Wk_t_ref[l]
        Wvt = Wv_t_ref[l]
        Wed = W_edge_ref[l]
        e2 = edge.reshape(_L * _DP, _LAT)
        e_kv = jnp.dot(e2, jnp.concatenate([Wk[:_LAT], Wv[:_LAT]], axis=1),
                       preferred_element_type=f32)
        tm3 = jnp.dot(
            tm,
            jnp.concatenate([Wk[_LAT:], Wv[_LAT:], Wqt], axis=1),
            preferred_element_type=f32)
        q = jnp.dot(ch, Wq, preferred_element_type=f32) * (1.0 / _SC)
        p = ((e_kv[:, :_LAT].reshape(_L, _DP, _LAT) + tm3[:, None, :_LAT])
             * q[None, :, :]).reshape(_L * _DP, _LAT)
        e_s = (jnp.exp(jnp.dot(p, BD, preferred_element_type=f32) + bias2)
               .reshape(_L, _DP, _LAT))
        r_s = 1.0 / (jnp.sum(e_s, axis=0) + 1e-30)
        o = jnp.sum(e_s * (e_kv[:, _LAT:].reshape(_L, _DP, _LAT)
                           + tm3[:, None, _LAT:2 * _LAT]),
                    axis=0) * r_s
        ch = ch + jax.nn.relu(jnp.dot(o, Wo, preferred_element_type=f32))

        Wot = Wo_t_ref[l]
        qt = tm3[:, 2 * _LAT:] * (1.0 / _SC)
        ch3 = jnp.dot(
            ch,
            jnp.concatenate([Wkt[_LAT:], Wvt[_LAT:], Wed[2 * _LAT:]], axis=1),
            preferred_element_type=f32)
        e_tt = jnp.dot(e2, jnp.concatenate([Wkt[:_LAT], Wvt[:_LAT]], axis=1),
                       preferred_element_type=f32)
        pt = ((e_tt[:, :_LAT].reshape(_L, _DP, _LAT) + ch3[None, :, :_LAT])
              * qt[:, None, :]).reshape(_L * _DP, _LAT)
        e_t = (jnp.exp(jnp.dot(pt, BD, preferred_element_type=f32) + bias2)
               .reshape(_L, _DP, _LAT))
        r_t = 1.0 / (jnp.sum(e_t, axis=1) + 1e-30)
        ot = jnp.sum(e_t * (e_tt[:, _LAT:].reshape(_L, _DP, _LAT)
                            + ch3[None, :, _LAT:2 * _LAT]),
                     axis=1) * r_t
        tm = tm + jax.nn.relu(jnp.dot(ot, Wot, preferred_element_type=f32))

        pre = (jnp.dot(e2, Wed[:_LAT], preferred_element_type=f32)
               .reshape(_L, _DP, _LAT)
               + jnp.dot(tm, Wed[_LAT:2 * _LAT],
                         preferred_element_type=f32)[:, None, :]
               + ch3[None, :, 2 * _LAT:])
        edge = edge + jax.nn.relu(pre)

    ohq = jnp.transpose(ohqT)
    edge_q = jnp.sum(edge[_LX:, :, :] * ohq[:, :, None], axis=1)
    tm_q = tm[_LX:, :]
    ch_q = jnp.dot(ohq, ch, preferred_element_type=f32)
    Wout = W_out_ref[...]
    hq = (jnp.dot(edge_q, Wout[:_LAT], preferred_element_type=f32)
          + jnp.dot(tm_q, Wout[_LAT:2 * _LAT], preferred_element_type=f32)
          + jnp.dot(ch_q, Wout[2 * _LAT:], preferred_element_type=f32))
    h_qry_ref[0] = hq

    ohx_m = jnp.transpose(ohxT * mx)
    h_obs = jnp.sum(edge[:_LX, :, :] * ohx_m[:, :, None], axis=1)
    h_obs_ref[0] = h_obs


def kernel(tx, cx, mx, x, tq, cq, mq, W_time, C_emb, W_e, Wq_c, Wk_c, Wv_c,
           Wo_c, Wq_t, Wk_t, Wv_t, Wo_t, W_edge, W_out):
    f32 = jnp.float32
    txr = tx.reshape(_B, 1, _LX).astype(f32)
    cxr = cx.reshape(_B, 1, _LX).astype(jnp.int32)
    mxr = mx.reshape(_B, 1, _LX).astype(f32)
    xr = x.reshape(_B, 1, _LX).astype(f32)
    tqr = tq.reshape(_B, 1, _LQ).astype(f32)
    cqr = cq.reshape(_B, 1, _LQ).astype(jnp.int32)
    mqr = mq.reshape(_B, 1, _LQ).astype(f32)
    C_emb_p = jnp.zeros((_DP, _LAT), f32).at[:_D].set(C_emb.astype(f32))

    def row_spec(n):
        return pl.BlockSpec((1, 1, n), lambda b: (b, 0, 0))

    def full_spec(arr):
        nd = arr.ndim
        return pl.BlockSpec(arr.shape, lambda b: (0,) * nd)

    weights = [W_time, C_emb_p, W_e, Wq_c, Wk_c, Wv_c, Wo_c,
               Wq_t, Wk_t, Wv_t, Wo_t, W_edge, W_out]
    in_specs = [row_spec(_LX), row_spec(_LX), row_spec(_LX), row_spec(_LX),
                row_spec(_LQ), row_spec(_LQ), row_spec(_LQ)]
    in_specs += [full_spec(w) for w in weights]

    out_shape = [jax.ShapeDtypeStruct((_B, _LX, _LAT), f32),
                 jax.ShapeDtypeStruct((_B, _LQ, _NG * _LAT), f32)]
    out_specs = [pl.BlockSpec((1, _LX, _LAT), lambda b: (b, 0, 0)),
                 pl.BlockSpec((1, _LQ, _NG * _LAT), lambda b: (b, 0, 0))]

    h_obs, hq = pl.pallas_call(
        _fwd_kernel,
        grid=(_B,),
        in_specs=in_specs,
        out_specs=out_specs,
        out_shape=out_shape,
        compiler_params=pltpu.CompilerParams(
            dimension_semantics=("parallel",),
            vmem_limit_bytes=128 * 1024 * 1024,
        ),
    )(txr, cxr, mxr, xr, tqr, cqr, mqr, *weights)

    h_qry = hq.reshape(_B, _LQ, _NG, _LAT).transpose(0, 2, 1, 3)
    return h_obs, h_qry
